# baseline (speedup 1.0000x reference)
import jax
import jax.numpy as jnp
from jax.experimental import pallas as pl

B = 8
C = 768
H = 32
W = 32
HALF = C // 2


HB = 16
STEPS = (B * H) // HB


def _fill_kernel(row_ref, col_ref, out_ref):
    h0 = (pl.program_id(0) * HB) % H
    col = col_ref[0:W, :]
    out_ref[0, :, :, 0:HALF] = jnp.broadcast_to(col[None], (HB, W, HALF))
    rows = row_ref[pl.ds(h0, HB), :]
    out_ref[0, :, :, HALF:C] = jnp.broadcast_to(
        rows[:, None, :], (HB, W, HALF))


@jax.jit
def _pos_embed(row_emb, col_emb):
    return pl.pallas_call(
        _fill_kernel,
        grid=(STEPS,),
        in_specs=[
            pl.BlockSpec(row_emb.shape, lambda i: (0, 0)),
            pl.BlockSpec(col_emb.shape, lambda i: (0, 0)),
        ],
        out_specs=pl.BlockSpec(
            (1, HB, W, C), lambda i: (i // (H // HB), i % (H // HB), 0, 0)),
        out_shape=jax.ShapeDtypeStruct((B, H, W, C), jnp.float32),
    )(row_emb, col_emb)


def kernel(x, row_emb, col_emb):
    out = _pos_embed(row_emb, col_emb)
    return jnp.transpose(out, (0, 3, 1, 2))


# speedup vs baseline: 1.2066x; 1.2066x over previous
import jax
import jax.numpy as jnp
from jax.experimental import pallas as pl

B = 8
C = 768
H = 32
W = 32
HALF = C // 2


BB = 2
STEPS = B // BB


def _fill_kernel(row_ref, col_ref, out_ref):
    col = col_ref[0:W, :]
    out_ref[:, :, :, 0:HALF] = jnp.broadcast_to(
        col[None, None], (BB, H, W, HALF))
    rows = row_ref[0:H, :]
    out_ref[:, :, :, HALF:C] = jnp.broadcast_to(
        rows[None, :, None, :], (BB, H, W, HALF))


@jax.jit
def _pos_embed(row_emb, col_emb):
    return pl.pallas_call(
        _fill_kernel,
        grid=(STEPS,),
        in_specs=[
            pl.BlockSpec(row_emb.shape, lambda i: (0, 0)),
            pl.BlockSpec(col_emb.shape, lambda i: (0, 0)),
        ],
        out_specs=pl.BlockSpec((BB, H, W, C), lambda i: (i, 0, 0, 0)),
        out_shape=jax.ShapeDtypeStruct((B, H, W, C), jnp.float32),
    )(row_emb, col_emb)


def kernel(x, row_emb, col_emb):
    out = _pos_embed(row_emb, col_emb)
    return jnp.transpose(out, (0, 3, 1, 2))


# speedup vs baseline: 1.2551x; 1.0402x over previous
import jax
import jax.numpy as jnp
from jax.experimental import pallas as pl
from jax.experimental.pallas import tpu as pltpu

B = 8
C = 768
H = 32
W = 32
HALF = C // 2


def _fill_kernel(row_ref, col_ref, out_ref, buf, sem):
    col = col_ref[0:W, :]
    buf[:, :, 0:HALF] = jnp.broadcast_to(col[None], (H, W, HALF))
    rows = row_ref[0:H, :]
    buf[:, :, HALF:C] = jnp.broadcast_to(rows[:, None, :], (H, W, HALF))
    copies = [
        pltpu.make_async_copy(buf, out_ref.at[b], sem) for b in range(B)
    ]
    for cp in copies:
        cp.start()
    for cp in copies:
        cp.wait()


@jax.jit
def _pos_embed(row_emb, col_emb):
    return pl.pallas_call(
        _fill_kernel,
        in_specs=[
            pl.BlockSpec(memory_space=pltpu.VMEM),
            pl.BlockSpec(memory_space=pltpu.VMEM),
        ],
        out_specs=pl.BlockSpec(memory_space=pl.ANY),
        out_shape=jax.ShapeDtypeStruct((B, H, W, C), jnp.float32),
        scratch_shapes=[
            pltpu.VMEM((H, W, C), jnp.float32),
            pltpu.SemaphoreType.DMA,
        ],
    )(row_emb, col_emb)


def kernel(x, row_emb, col_emb):
    out = _pos_embed(row_emb, col_emb)
    return jnp.transpose(out, (0, 3, 1, 2))


# speedup vs baseline: 1.2643x; 1.0073x over previous
import jax
import jax.numpy as jnp
from jax.experimental import pallas as pl

B = 8
C = 768
H = 32
W = 32
HALF = C // 2


def _fill_kernel(row_ref, col_ref, out_ref):
    col = col_ref[0:W, :]
    out_ref[0, :, :, 0:HALF] = jnp.broadcast_to(col[None], (H, W, HALF))
    rows = row_ref[0:H, :]
    out_ref[0, :, :, HALF:C] = jnp.broadcast_to(
        rows[:, None, :], (H, W, HALF))


@jax.jit
def _pos_embed(row_emb, col_emb):
    return pl.pallas_call(
        _fill_kernel,
        grid=(B,),
        in_specs=[
            pl.BlockSpec(row_emb.shape, lambda i: (0, 0)),
            pl.BlockSpec(col_emb.shape, lambda i: (0, 0)),
        ],
        out_specs=pl.BlockSpec((1, H, W, C), lambda i: (i, 0, 0, 0)),
        out_shape=jax.ShapeDtypeStruct((B, H, W, C), jnp.float32),
    )(row_emb, col_emb)


def kernel(x, row_emb, col_emb):
    out = _pos_embed(row_emb, col_emb)
    return jnp.transpose(out, (0, 3, 1, 2))
